# baseline (speedup 1.0000x reference)
import functools

import jax
import jax.numpy as jnp
from jax import lax
from jax.experimental import pallas as pl
from jax.experimental.pallas import tpu as pltpu
from jax.experimental.pallas import tpu_sc as plsc

IDXG = 32
NBUF = 4


def _make_lookup(B: int, L: int, D: int):
    info = plsc.get_sparse_core_info()
    NC, NS = info.num_cores, info.num_subcores
    NW = NC * NS
    assert B % (NW * IDXG) == 0
    b_per_w = B // NW
    n_groups = b_per_w // IDXG

    mesh = plsc.VectorSubcoreMesh(core_axis_name="c", subcore_axis_name="s")

    @functools.partial(
        pl.kernel,
        mesh=mesh,
        out_type=jax.ShapeDtypeStruct((B, L, D), jnp.float32),
        scratch_types=[
            pltpu.VMEM((IDXG * L,), jnp.int32),
            [pltpu.VMEM((L, D), jnp.float32) for _ in range(NBUF)],
            [pltpu.SemaphoreType.DMA for _ in range(NBUF)],
            [pltpu.SemaphoreType.DMA for _ in range(NBUF)],
        ],
        compiler_params=pltpu.CompilerParams(use_tc_tiling_on_sc=False),
    )
    def lookup(idx_hbm, table_hbm, out_hbm, idx_v, rows, gsem, ssem):
        wid = lax.axis_index("s") * NC + lax.axis_index("c")
        base_b = wid * b_per_w

        def gather(k):
            src = table_hbm.at[idx_v.at[pl.ds(k * L, L)]]
            return pltpu.make_async_copy(src, rows[k % NBUF],
                                         gsem[k % NBUF])

        def store(g, k):
            b = base_b + g * IDXG + k
            return pltpu.make_async_copy(rows[k % NBUF], out_hbm.at[b],
                                         ssem[k % NBUF])

        def body(g, _):
            off = (base_b + g * IDXG) * L
            pltpu.sync_copy(idx_hbm.at[pl.ds(off, IDXG * L)], idx_v)
            for k in range(IDXG):
                if k >= NBUF:
                    store(g, k - NBUF).wait()
                gather(k).start()
                if k >= 1:
                    gather(k - 1).wait()
                    store(g, k - 1).start()
            gather(IDXG - 1).wait()
            store(g, IDXG - 1).start()
            for k in range(IDXG - NBUF, IDXG):
                store(g, k).wait()
            return ()

        lax.fori_loop(0, n_groups, body, (), unroll=False)

    return lookup


def kernel(p_sequences, table):
    B, L = p_sequences.shape
    V, D = table.shape
    idx_flat = p_sequences.reshape(B * L)
    lookup = _make_lookup(B, L, D)
    return lookup(idx_flat, table)
